# baseline (speedup 1.0000x reference)
import functools

import numpy as np
import jax
import jax.numpy as jnp
from jax import lax
from jax.experimental import pallas as pl
from jax.experimental.pallas import tpu as pltpu
from jax.experimental.pallas import tpu_sc as plsc

_B, _N, _C = 32, 5000, 91
_NB = 5000
_NVALID = 61
_HIBITS = int(np.asarray(1e30, np.float32).view(np.int32))
_M = 128
_S1 = 2048
_S2 = 256
_OUTP = 112


def _prob_kernel(logits_ref, obj_ref, unk_ref, prob_ref, v_ref):
    l = logits_ref[0]
    e = jnp.exp(-l[:, 63:64])
    u = jax.nn.sigmoid(l[:, 64:65])
    s = jax.nn.sigmoid(l)
    p = (e * s) * (1.0 - u)
    punk = e * u
    zeros = jnp.zeros((l.shape[0], 66), jnp.float32)
    out = jnp.concatenate([p[:, :_NVALID], punk, zeros], axis=1)
    prob_ref[0] = out
    v_ref[0] = jnp.max(out, axis=1, keepdims=True)


def _compute_prob64(pred_logits, pred_obj, pred_unk):
    B, N, C = pred_logits.shape
    obj3 = pred_obj[..., None]
    unk3 = pred_unk[..., None]
    grid = (B, N // _NB)
    return pl.pallas_call(
        _prob_kernel,
        grid=grid,
        in_specs=[
            pl.BlockSpec((1, _NB, C), lambda b, n: (b, n, 0)),
            pl.BlockSpec((1, _NB, 1), lambda b, n: (b, n, 0)),
            pl.BlockSpec((1, _NB, 1), lambda b, n: (b, n, 0)),
        ],
        out_specs=[
            pl.BlockSpec((1, _NB, 128), lambda b, n: (b, n, 0)),
            pl.BlockSpec((1, _NB, 1), lambda b, n: (b, n, 0)),
        ],
        out_shape=[
            jax.ShapeDtypeStruct((B, N, 128), jnp.float32),
            jax.ShapeDtypeStruct((B, N, 1), jnp.float32),
        ],
        compiler_params=pltpu.CompilerParams(
            dimension_semantics=("parallel", "arbitrary")),
    )(pred_logits, obj3, unk3)


def _thresh_kernel(v_ref, th_ref, thf_ref):
    bits = lax.bitcast_convert_type(v_ref[...], jnp.int32)
    B = bits.shape[0]
    lo0 = jnp.zeros((B, 1), jnp.int32)
    hi0 = jnp.full((B, 1), _HIBITS, jnp.int32)

    def body(_, c):
        lo, hi = c
        mid = lo + ((hi - lo) >> 1)
        cnt = jnp.sum((bits > mid).astype(jnp.int32), axis=1, keepdims=True)
        take = cnt >= 100
        return jnp.where(take, mid, lo), jnp.where(take, hi, mid)

    lo, _ = lax.fori_loop(0, 30, body, (lo0, hi0))
    th_ref[...] = jnp.broadcast_to(lo, (B, 16))
    thf_ref[...] = jnp.broadcast_to(
        lax.bitcast_convert_type(lo, jnp.float32), (B, 16))


def _compute_thresh(v2):
    B = v2.shape[0]
    return pl.pallas_call(
        _thresh_kernel,
        out_shape=[jax.ShapeDtypeStruct((B, 16), jnp.int32),
                   jax.ShapeDtypeStruct((B, 16), jnp.float32)],
    )(v2)


def _make_sc_select():
    mesh = plsc.VectorSubcoreMesh(
        core_axis_name="c", subcore_axis_name="s", num_cores=2, num_subcores=16)
    return functools.partial(
        pl.kernel,
        out_type=[
            jax.ShapeDtypeStruct((_B * _OUTP,), jnp.float32),
            jax.ShapeDtypeStruct((_B * _OUTP,), jnp.int32),
            jax.ShapeDtypeStruct((_B * _OUTP * 4,), jnp.float32),
        ],
        mesh=mesh,
        compiler_params=pltpu.CompilerParams(needs_layout_passes=False),
        scratch_types=[
        pltpu.VMEM((5056,), jnp.float32),
        pltpu.VMEM((16,), jnp.int32),
        pltpu.VMEM((16,), jnp.float32),
        pltpu.VMEM((32,), jnp.float32),
        pltpu.VMEM((_M,), jnp.int32),
        pltpu.VMEM((_M, 128), jnp.float32),
        pltpu.VMEM((_S1,), jnp.float32),
        pltpu.VMEM((_S1,), jnp.int32),
        pltpu.VMEM((_S2,), jnp.float32),
        pltpu.VMEM((_S2,), jnp.int32),
        pltpu.VMEM((128,), jnp.int32),
        pltpu.VMEM((20000,), jnp.float32),
        pltpu.VMEM((128,), jnp.float32),
        pltpu.VMEM((128,), jnp.int32),
        pltpu.VMEM((_OUTP * 4,), jnp.float32),
            pltpu.SemaphoreType.DMA,
            pltpu.SemaphoreType.DMA,
        ],
    )


def _sc_body(v_hbm, th_hbm, thf_hbm, prob_hbm, boxes_hbm, scale_hbm,
               scores_hbm, labels_hbm, boxf_hbm,
               vrow, thbuf, thfbuf, scbuf, selidx, probbuf, sval, sidx,
               s2val, s2idx, anchoro, boxbuf, scoreo, labelo, boxo,
               sem, semb):
    b = lax.axis_index("s") * 2 + lax.axis_index("c")
    iota = lax.iota(jnp.int32, 16)
    zi = jnp.zeros((16,), jnp.int32)
    zf = jnp.zeros((16,), jnp.float32)

    boxcp = pltpu.make_async_copy(
        boxes_hbm.at[pl.ds(b * (_N * 4), _N * 4)], boxbuf, semb)
    boxcp.start()

    vrow[pl.ds(4992, 16)] = zf
    pltpu.sync_copy(v_hbm.at[pl.ds(b * _N, _N)], vrow.at[pl.ds(0, 5000)])
    pltpu.sync_copy(th_hbm.at[pl.ds(b * 16, 16)], thbuf)
    pltpu.sync_copy(thf_hbm.at[pl.ds(b * 16, 16)], thfbuf)
    pltpu.sync_copy(scale_hbm.at[pl.ds(b * 32, 32)], scbuf)

    lof = thfbuf[...]
    base = b * _N

    for g in range(_M // 16):
        selidx[pl.ds(g * 16, 16)] = zi

    vrow[pl.ds(5008, 16)] = zf
    vrow[pl.ds(5024, 16)] = zf
    vrow[pl.ds(5040, 16)] = zf

    def comp(j, cnt):
        for k in range(4):
            g = j * 4 + k
            m = vrow[pl.ds(g * 16, 16)] > lof
            mi = m.astype(jnp.int32)
            pos = cnt + plsc.cumsum(mi) - 1
            plsc.store_scatter(selidx, [pos], base + g * 16 + iota,
                               mask=m & (pos < _M))
            cnt = cnt + jnp.sum(mi)
        return cnt

    cnta = lax.fori_loop(0, 79, comp, jnp.int32(0))
    cnta_s = jnp.minimum(cnta, _M)

    pltpu.async_copy(prob_hbm.at[selidx], probbuf, sem).wait()

    def zrow(r, _):
        rv = zi + r
        for c4 in range(4):
            plsc.store_scatter(probbuf, [rv, c4 * 16 + iota], zf)
        return 0

    lax.fori_loop(cnta_s, _M, zrow, 0)

    def pre(i, _):
        sval[pl.ds(i * 16, 16)] = zf - 1.0
        return 0

    lax.fori_loop(0, _S1 // 16, pre, 0)

    def surv(j, cnt):
        for k in range(4):
            jv = (j * 4 + k) * 16 + iota
            slot = jv >> 6
            cc = jv & 63
            val = plsc.load_gather(probbuf, [slot, cc])
            m = val > lof
            anc = plsc.load_gather(selidx, [slot]) - base
            orig = anc * _C + jnp.where(cc == _NVALID, _C - 1, cc)
            mi = m.astype(jnp.int32)
            pos = cnt + plsc.cumsum(mi) - 1
            m2 = m & (pos < _S1)
            plsc.store_scatter(sval, [pos], val, mask=m2)
            plsc.store_scatter(sidx, [pos], orig, mask=m2)
            cnt = cnt + jnp.sum(mi)
        return cnt

    cnt1 = lax.fori_loop(0, (_M * 64) // 64, surv, jnp.int32(0))
    g1 = (jnp.minimum(cnt1, _S1) + 15) >> 4

    g1u = (g1 + 3) >> 2

    def count_sv(midf):
        midv = zf + midf

        def cb(i, c):
            for k in range(4):
                x = sval[pl.ds((i * 4 + k) * 16, 16)]
                c = c + jnp.where(x > midv, 1, 0).astype(jnp.int32)
            return c

        return jnp.sum(lax.fori_loop(0, g1u, cb, zi))

    def bis(_, c):
        lo3, hi3 = c
        mid = lo3 + ((hi3 - lo3) >> 1)
        midf = lax.bitcast_convert_type(mid, jnp.float32)
        take = count_sv(midf) >= 100
        return jnp.where(take, mid, lo3), jnp.where(take, hi3, mid)

    lo3_i, _hi3 = lax.fori_loop(
        0, 30, bis, (jnp.max(thbuf[...]), jnp.int32(_HIBITS)))
    lo3f = zf + lax.bitcast_convert_type(lo3_i, jnp.float32)

    for g in range(_S2 // 16):
        s2val[pl.ds(g * 16, 16)] = zf - 1.0

    def surv2(i, cnt):
        for k in range(4):
            val = sval[pl.ds((i * 4 + k) * 16, 16)]
            idxv = sidx[pl.ds((i * 4 + k) * 16, 16)]
            m = val > lo3f
            mi = m.astype(jnp.int32)
            pos = cnt + plsc.cumsum(mi) - 1
            m2 = m & (pos < _S2)
            plsc.store_scatter(s2val, [pos], val, mask=m2)
            plsc.store_scatter(s2idx, [pos], idxv, mask=m2)
            cnt = cnt + jnp.sum(mi)
        return cnt

    cnt3 = lax.fori_loop(0, g1u, surv2, jnp.int32(0))

    for g in range(_OUTP // 16):
        scoreo[pl.ds(g * 16, 16)] = zf
        labelo[pl.ds(g * 16, 16)] = zi
        anchoro[pl.ds(g * 16, 16)] = zi
    lane0 = iota == 0
    big = jnp.int32(2 ** 30)

    _NV = _S2 // 16

    def sel(i, carry):
        vals = carry[:_NV]
        ids = carry[_NV:]
        mv = vals[0]
        for k in range(1, _NV):
            mv = jnp.maximum(mv, vals[k])
        ms = jnp.max(mv)
        msv = zf + ms
        iv = jnp.where(vals[0] == msv, ids[0], big)
        for k in range(1, _NV):
            iv = jnp.minimum(iv, jnp.where(vals[k] == msv, ids[k], big))
        imin = jnp.min(iv)
        iminv = zi + imin
        newvals = tuple(
            jnp.where((vals[k] == msv) & (ids[k] == iminv), -2.0, vals[k])
            for k in range(_NV))
        posi = zi + i
        plsc.store_scatter(scoreo, [posi], msv, mask=lane0)
        plsc.store_scatter(labelo, [posi], iminv % _C, mask=lane0)
        plsc.store_scatter(anchoro, [posi], iminv // _C, mask=lane0)
        return newvals + ids

    init = tuple(s2val[pl.ds(k * 16, 16)] for k in range(_NV)) + tuple(
        s2idx[pl.ds(k * 16, 16)] for k in range(_NV))
    lax.fori_loop(0, 100, sel, init)

    boxcp.wait()
    W = scbuf[pl.ds(0, 16)]
    H = scbuf[pl.ds(16, 16)]
    for r in range(_OUTP // 16):
        av4 = anchoro[pl.ds(r * 16, 16)] * 4
        cx = plsc.load_gather(boxbuf, [av4])
        cy = plsc.load_gather(boxbuf, [av4 + 1])
        w = plsc.load_gather(boxbuf, [av4 + 2])
        h = plsc.load_gather(boxbuf, [av4 + 3])
        x0 = (cx - 0.5 * w) * W
        y0 = (cy - 0.5 * h) * H
        x1 = (cx + 0.5 * w) * W
        y1 = (cy + 0.5 * h) * H
        bpos = (r * 16 + iota) * 4
        plsc.store_scatter(boxo, [bpos], x0)
        plsc.store_scatter(boxo, [bpos + 1], y0)
        plsc.store_scatter(boxo, [bpos + 2], x1)
        plsc.store_scatter(boxo, [bpos + 3], y1)

    pltpu.sync_copy(scoreo.at[pl.ds(0, _OUTP)],
                    scores_hbm.at[pl.ds(b * _OUTP, _OUTP)])
    pltpu.sync_copy(labelo.at[pl.ds(0, _OUTP)],
                    labels_hbm.at[pl.ds(b * _OUTP, _OUTP)])
    pltpu.sync_copy(boxo, boxf_hbm.at[pl.ds(b * (_OUTP * 4), _OUTP * 4)])


@functools.lru_cache(maxsize=None)
def _get_sc_select():
    return _make_sc_select()(_sc_body)


def kernel(pred_logits, pred_obj, pred_boxes, pred_unk, target_sizes):
    B, N, C = pred_logits.shape
    prob64, v3 = _compute_prob64(pred_logits, pred_obj, pred_unk)
    v2 = v3[..., 0]
    th, thf = _compute_thresh(v2)
    ts = target_sizes.astype(jnp.float32)
    scale32 = jnp.concatenate(
        [jnp.broadcast_to(ts[:, 1:2], (B, 16)),
         jnp.broadcast_to(ts[:, 0:1], (B, 16))], axis=1)
    prob_flat = prob64.reshape(B * N, 128)
    scores_p, labels_p, boxf = _get_sc_select()(
        v2.reshape(-1), th.reshape(-1), thf.reshape(-1), prob_flat,
        pred_boxes.reshape(-1), scale32.reshape(-1))
    scores = scores_p.reshape(B, _OUTP)[:, :100]
    labels = labels_p.reshape(B, _OUTP)[:, :100]
    boxes = boxf.reshape(B, _OUTP, 4)[:, :100, :]
    return scores, labels, boxes


# speedup vs baseline: 1.0111x; 1.0111x over previous
import functools

import numpy as np
import jax
import jax.numpy as jnp
from jax import lax
from jax.experimental import pallas as pl
from jax.experimental.pallas import tpu as pltpu
from jax.experimental.pallas import tpu_sc as plsc

_B, _N, _C = 32, 5000, 91
_NB = 5000
_NVALID = 61
_HIBITS = int(np.asarray(1e30, np.float32).view(np.int32))
_M = 128
_S1 = 2048
_S2 = 256
_OUTP = 112


def _prob_kernel(logits_ref, obj_ref, unk_ref, prob_ref, v_ref):
    l = logits_ref[0]
    e = jnp.exp(-obj_ref[0])
    u = jax.nn.sigmoid(unk_ref[0])
    s = jax.nn.sigmoid(l)
    p = (e * s) * (1.0 - u)
    punk = e * u
    zeros = jnp.zeros((l.shape[0], 66), jnp.float32)
    out = jnp.concatenate([p[:, :_NVALID], punk, zeros], axis=1)
    prob_ref[0] = out
    v_ref[0] = jnp.max(out, axis=1, keepdims=True)


def _compute_prob64(pred_logits, pred_obj, pred_unk):
    B, N, C = pred_logits.shape
    obj3 = pred_obj[..., None]
    unk3 = pred_unk[..., None]
    grid = (B, N // _NB)
    return pl.pallas_call(
        _prob_kernel,
        grid=grid,
        in_specs=[
            pl.BlockSpec((1, _NB, C), lambda b, n: (b, n, 0)),
            pl.BlockSpec((1, _NB, 1), lambda b, n: (b, n, 0)),
            pl.BlockSpec((1, _NB, 1), lambda b, n: (b, n, 0)),
        ],
        out_specs=[
            pl.BlockSpec((1, _NB, 128), lambda b, n: (b, n, 0)),
            pl.BlockSpec((1, _NB, 1), lambda b, n: (b, n, 0)),
        ],
        out_shape=[
            jax.ShapeDtypeStruct((B, N, 128), jnp.float32),
            jax.ShapeDtypeStruct((B, N, 1), jnp.float32),
        ],
        compiler_params=pltpu.CompilerParams(
            dimension_semantics=("parallel", "arbitrary")),
    )(pred_logits, obj3, unk3)


def _thresh_kernel(v_ref, th_ref, thf_ref):
    bits = lax.bitcast_convert_type(v_ref[...], jnp.int32)
    B = bits.shape[0]
    lo0 = jnp.zeros((B, 1), jnp.int32)
    hi0 = jnp.full((B, 1), _HIBITS, jnp.int32)

    def body(_, c):
        lo, hi = c
        mid = lo + ((hi - lo) >> 1)
        cnt = jnp.sum((bits > mid).astype(jnp.int32), axis=1, keepdims=True)
        take = cnt >= 100
        return jnp.where(take, mid, lo), jnp.where(take, hi, mid)

    lo, _ = lax.fori_loop(0, 30, body, (lo0, hi0))
    th_ref[...] = jnp.broadcast_to(lo, (B, 16))
    thf_ref[...] = jnp.broadcast_to(
        lax.bitcast_convert_type(lo, jnp.float32), (B, 16))


def _compute_thresh(v2):
    B = v2.shape[0]
    return pl.pallas_call(
        _thresh_kernel,
        out_shape=[jax.ShapeDtypeStruct((B, 16), jnp.int32),
                   jax.ShapeDtypeStruct((B, 16), jnp.float32)],
    )(v2)


def _make_sc_select():
    mesh = plsc.VectorSubcoreMesh(
        core_axis_name="c", subcore_axis_name="s", num_cores=2, num_subcores=16)
    return functools.partial(
        pl.kernel,
        out_type=[
            jax.ShapeDtypeStruct((_B * _OUTP,), jnp.float32),
            jax.ShapeDtypeStruct((_B * _OUTP,), jnp.int32),
            jax.ShapeDtypeStruct((_B * _OUTP * 4,), jnp.float32),
        ],
        mesh=mesh,
        compiler_params=pltpu.CompilerParams(needs_layout_passes=False),
        scratch_types=[
        pltpu.VMEM((5056,), jnp.float32),
        pltpu.VMEM((16,), jnp.int32),
        pltpu.VMEM((16,), jnp.float32),
        pltpu.VMEM((32,), jnp.float32),
        pltpu.VMEM((_M,), jnp.int32),
        pltpu.VMEM((_M, 128), jnp.float32),
        pltpu.VMEM((_S1,), jnp.float32),
        pltpu.VMEM((_S1,), jnp.int32),
        pltpu.VMEM((_S2,), jnp.float32),
        pltpu.VMEM((_S2,), jnp.int32),
        pltpu.VMEM((128,), jnp.int32),
        pltpu.VMEM((20000,), jnp.float32),
        pltpu.VMEM((128,), jnp.float32),
        pltpu.VMEM((128,), jnp.int32),
        pltpu.VMEM((_OUTP * 4,), jnp.float32),
            pltpu.SemaphoreType.DMA,
            pltpu.SemaphoreType.DMA,
        ],
    )


def _sc_body(v_hbm, th_hbm, thf_hbm, prob_hbm, boxes_hbm, scale_hbm,
               scores_hbm, labels_hbm, boxf_hbm,
               vrow, thbuf, thfbuf, scbuf, selidx, probbuf, sval, sidx,
               s2val, s2idx, anchoro, boxbuf, scoreo, labelo, boxo,
               sem, semb):
    b = lax.axis_index("s") * 2 + lax.axis_index("c")
    iota = lax.iota(jnp.int32, 16)
    zi = jnp.zeros((16,), jnp.int32)
    zf = jnp.zeros((16,), jnp.float32)

    boxcp = pltpu.make_async_copy(
        boxes_hbm.at[pl.ds(b * (_N * 4), _N * 4)], boxbuf, semb)
    boxcp.start()

    vrow[pl.ds(4992, 16)] = zf
    pltpu.sync_copy(v_hbm.at[pl.ds(b * _N, _N)], vrow.at[pl.ds(0, 5000)])
    pltpu.sync_copy(th_hbm.at[pl.ds(b * 16, 16)], thbuf)
    pltpu.sync_copy(thf_hbm.at[pl.ds(b * 16, 16)], thfbuf)
    pltpu.sync_copy(scale_hbm.at[pl.ds(b * 32, 32)], scbuf)

    lof = thfbuf[...]
    base = b * _N

    for g in range(_M // 16):
        selidx[pl.ds(g * 16, 16)] = zi

    vrow[pl.ds(5008, 16)] = zf
    vrow[pl.ds(5024, 16)] = zf
    vrow[pl.ds(5040, 16)] = zf

    def comp(j, cnt):
        for k in range(4):
            g = j * 4 + k
            m = vrow[pl.ds(g * 16, 16)] > lof
            mi = m.astype(jnp.int32)
            pos = cnt + plsc.cumsum(mi) - 1
            plsc.store_scatter(selidx, [pos], base + g * 16 + iota,
                               mask=m & (pos < _M))
            cnt = cnt + jnp.sum(mi)
        return cnt

    cnta = lax.fori_loop(0, 79, comp, jnp.int32(0))
    cnta_s = jnp.minimum(cnta, _M)

    pltpu.async_copy(prob_hbm.at[selidx], probbuf, sem).wait()

    def zrow(r, _):
        rv = zi + r
        for c4 in range(4):
            plsc.store_scatter(probbuf, [rv, c4 * 16 + iota], zf)
        return 0

    lax.fori_loop(cnta_s, _M, zrow, 0)

    def pre(i, _):
        sval[pl.ds(i * 16, 16)] = zf - 1.0
        return 0

    lax.fori_loop(0, _S1 // 16, pre, 0)

    def surv(j, cnt):
        for k in range(4):
            jv = (j * 4 + k) * 16 + iota
            slot = jv >> 6
            cc = jv & 63
            val = plsc.load_gather(probbuf, [slot, cc])
            m = val > lof
            anc = plsc.load_gather(selidx, [slot]) - base
            orig = anc * _C + jnp.where(cc == _NVALID, _C - 1, cc)
            mi = m.astype(jnp.int32)
            pos = cnt + plsc.cumsum(mi) - 1
            m2 = m & (pos < _S1)
            plsc.store_scatter(sval, [pos], val, mask=m2)
            plsc.store_scatter(sidx, [pos], orig, mask=m2)
            cnt = cnt + jnp.sum(mi)
        return cnt

    cnt1 = lax.fori_loop(0, (_M * 64) // 64, surv, jnp.int32(0))
    g1 = (jnp.minimum(cnt1, _S1) + 15) >> 4

    g1u = (g1 + 3) >> 2

    def count_sv(midf):
        midv = zf + midf

        def cb(i, c):
            for k in range(4):
                x = sval[pl.ds((i * 4 + k) * 16, 16)]
                c = c + jnp.where(x > midv, 1, 0).astype(jnp.int32)
            return c

        return jnp.sum(lax.fori_loop(0, g1u, cb, zi))

    def bis_cond(c):
        lo3, hi3, ccur, it = c
        return (ccur > _S2) & (it < jnp.int32(30))

    def bis(c):
        lo3, hi3, ccur, it = c
        mid = lo3 + ((hi3 - lo3) >> 1)
        midf = lax.bitcast_convert_type(mid, jnp.float32)
        cm = count_sv(midf)
        take = cm >= 100
        return (jnp.where(take, mid, lo3), jnp.where(take, hi3, mid),
                jnp.where(take, cm, ccur), it + 1)

    lo3_i, _hi3, _c3, _it = lax.while_loop(
        bis_cond, bis,
        (jnp.max(thbuf[...]), jnp.int32(_HIBITS), cnt1, jnp.int32(0)))
    lo3f = zf + lax.bitcast_convert_type(lo3_i, jnp.float32)

    for g in range(_S2 // 16):
        s2val[pl.ds(g * 16, 16)] = zf - 1.0

    def surv2(i, cnt):
        for k in range(4):
            val = sval[pl.ds((i * 4 + k) * 16, 16)]
            idxv = sidx[pl.ds((i * 4 + k) * 16, 16)]
            m = val > lo3f
            mi = m.astype(jnp.int32)
            pos = cnt + plsc.cumsum(mi) - 1
            m2 = m & (pos < _S2)
            plsc.store_scatter(s2val, [pos], val, mask=m2)
            plsc.store_scatter(s2idx, [pos], idxv, mask=m2)
            cnt = cnt + jnp.sum(mi)
        return cnt

    cnt3 = lax.fori_loop(0, g1u, surv2, jnp.int32(0))

    for g in range(_OUTP // 16):
        scoreo[pl.ds(g * 16, 16)] = zf
        labelo[pl.ds(g * 16, 16)] = zi
        anchoro[pl.ds(g * 16, 16)] = zi
    lane0 = iota == 0
    big = jnp.int32(2 ** 30)

    _NV = _S2 // 16

    def sel(i, carry):
        vals = carry[:_NV]
        ids = carry[_NV:]
        mv = vals[0]
        for k in range(1, _NV):
            mv = jnp.maximum(mv, vals[k])
        ms = jnp.max(mv)
        msv = zf + ms
        iv = jnp.where(vals[0] == msv, ids[0], big)
        for k in range(1, _NV):
            iv = jnp.minimum(iv, jnp.where(vals[k] == msv, ids[k], big))
        imin = jnp.min(iv)
        iminv = zi + imin
        newvals = tuple(
            jnp.where((vals[k] == msv) & (ids[k] == iminv), -2.0, vals[k])
            for k in range(_NV))
        posi = zi + i
        plsc.store_scatter(scoreo, [posi], msv, mask=lane0)
        plsc.store_scatter(labelo, [posi], iminv % _C, mask=lane0)
        plsc.store_scatter(anchoro, [posi], iminv // _C, mask=lane0)
        return newvals + ids

    init = tuple(s2val[pl.ds(k * 16, 16)] for k in range(_NV)) + tuple(
        s2idx[pl.ds(k * 16, 16)] for k in range(_NV))
    lax.fori_loop(0, 100, sel, init)

    boxcp.wait()
    W = scbuf[pl.ds(0, 16)]
    H = scbuf[pl.ds(16, 16)]
    for r in range(_OUTP // 16):
        av4 = anchoro[pl.ds(r * 16, 16)] * 4
        cx = plsc.load_gather(boxbuf, [av4])
        cy = plsc.load_gather(boxbuf, [av4 + 1])
        w = plsc.load_gather(boxbuf, [av4 + 2])
        h = plsc.load_gather(boxbuf, [av4 + 3])
        x0 = (cx - 0.5 * w) * W
        y0 = (cy - 0.5 * h) * H
        x1 = (cx + 0.5 * w) * W
        y1 = (cy + 0.5 * h) * H
        bpos = (r * 16 + iota) * 4
        plsc.store_scatter(boxo, [bpos], x0)
        plsc.store_scatter(boxo, [bpos + 1], y0)
        plsc.store_scatter(boxo, [bpos + 2], x1)
        plsc.store_scatter(boxo, [bpos + 3], y1)

    pltpu.sync_copy(scoreo.at[pl.ds(0, _OUTP)],
                    scores_hbm.at[pl.ds(b * _OUTP, _OUTP)])
    pltpu.sync_copy(labelo.at[pl.ds(0, _OUTP)],
                    labels_hbm.at[pl.ds(b * _OUTP, _OUTP)])
    pltpu.sync_copy(boxo, boxf_hbm.at[pl.ds(b * (_OUTP * 4), _OUTP * 4)])


@functools.lru_cache(maxsize=None)
def _get_sc_select():
    return _make_sc_select()(_sc_body)


def kernel(pred_logits, pred_obj, pred_boxes, pred_unk, target_sizes):
    B, N, C = pred_logits.shape
    prob64, v3 = _compute_prob64(pred_logits, pred_obj, pred_unk)
    v2 = v3[..., 0]
    th, thf = _compute_thresh(v2)
    ts = target_sizes.astype(jnp.float32)
    scale32 = jnp.concatenate(
        [jnp.broadcast_to(ts[:, 1:2], (B, 16)),
         jnp.broadcast_to(ts[:, 0:1], (B, 16))], axis=1)
    prob_flat = prob64.reshape(B * N, 128)
    scores_p, labels_p, boxf = _get_sc_select()(
        v2.reshape(-1), th.reshape(-1), thf.reshape(-1), prob_flat,
        pred_boxes.reshape(-1), scale32.reshape(-1))
    scores = scores_p.reshape(B, _OUTP)[:, :100]
    labels = labels_p.reshape(B, _OUTP)[:, :100]
    boxes = boxf.reshape(B, _OUTP, 4)[:, :100, :]
    return scores, labels, boxes
